# baseline (speedup 1.0000x reference)
import functools

import numpy as np
import jax
import jax.numpy as jnp
from jax import lax
from jax.experimental import pallas as pl
from jax.experimental.pallas import tpu as pltpu
from jax.experimental.pallas import tpu_sc as plsc

LANES = 16
NUM_CORES = 2
NUM_SUBCORES = 16
NUM_WORKERS = NUM_CORES * NUM_SUBCORES


def _positional_encoding(seq_len: int, d_model: int) -> np.ndarray:
    pos = np.arange(seq_len)[:, np.newaxis]
    i = np.arange(d_model)[np.newaxis, :]
    angle_rates = 1.0 / np.power(10000, 2 * (i // 2) / np.float32(d_model))
    angle_rads = pos * angle_rates
    angle_rads[:, 0::2] = np.sin(angle_rads[:, 0::2])
    angle_rads[:, 1::2] = np.cos(angle_rads[:, 1::2])
    return np.asarray(angle_rads, dtype=np.float32)


@jax.jit
def _sc_embed(table, idx, pos):
    S, D = pos.shape
    V, _ = table.shape
    B = idx.size // S
    SPW = S // NUM_WORKERS
    HP = SPW // 2
    NQ = B // 2
    scale = float(np.sqrt(np.float32(D)))

    mesh = plsc.VectorSubcoreMesh(core_axis_name="c", subcore_axis_name="s")

    @functools.partial(
        pl.kernel,
        mesh=mesh,
        out_type=jax.ShapeDtypeStruct((B * S, D), jnp.float32),
        scratch_types=[
            pltpu.VMEM((B, S), jnp.int32),
            pltpu.VMEM((SPW, D), jnp.float32),
        ] + [pltpu.VMEM((HP, D), jnp.float32)] * 8
          + [pltpu.SemaphoreType.DMA] * 16,
    )
    def k(table_hbm, idx_hbm, pos_hbm, out_hbm, idx_v, pos_v, *bufs_and_sems):
        buf = bufs_and_sems[:8]
        gs = bufs_and_sems[8:16]
        ws = bufs_and_sems[16:24]
        wid = lax.axis_index("s") * NUM_CORES + lax.axis_index("c")
        s0 = wid * SPW

        pltpu.sync_copy(idx_hbm, idx_v)

        def idx_at(q, j):
            return idx_v.at[4 * (q // 2) + j, pl.ds(s0 + (q % 2) * HP, HP)]

        def out_rows(b, h):
            return out_hbm.at[pl.ds(b * S + s0 + h * HP, HP)]

        for j in range(4):
            pltpu.async_copy(table_hbm.at[idx_at(0, j)], buf[j], gs[j])
        pltpu.sync_copy(pos_hbm.at[pl.ds(s0, SPW)], pos_v)

        def do_quad(qq, u):
            q = 2 * qq + u
            v = 1 - u
            b0 = 4 * qq
            for j in range(4):
                pltpu.make_async_copy(table_hbm.at[idx_at(q, j)],
                                      buf[4 * u + j], gs[4 * u + j]).wait()
            @pl.when(q >= 1)
            def _():
                pb0 = 4 * ((q - 1) // 2)
                for j in range(4):
                    pltpu.make_async_copy(
                        buf[4 * v + j], out_rows(pb0 + j, v), ws[4 * v + j]
                    ).wait()
            @pl.when(q + 1 < NQ)
            def _():
                for j in range(4):
                    pltpu.async_copy(table_hbm.at[idx_at(q + 1, j)],
                                     buf[4 * v + j], gs[4 * v + j])

            def per_row(r, c):
                for kk in range(D // LANES):
                    sl = pl.ds(kk * LANES, LANES)
                    pv = pos_v[u * HP + r, sl]
                    for j in range(4):
                        bj = buf[4 * u + j]
                        bj[r, sl] = bj[r, sl] * scale + pv
                return c

            lax.fori_loop(0, HP, per_row, 0)
            for j in range(4):
                pltpu.async_copy(buf[4 * u + j], out_rows(b0 + j, u),
                                 ws[4 * u + j])

        def quad2(qq, c):
            do_quad(qq, 0)
            do_quad(qq, 1)
            return c

        lax.fori_loop(0, NQ // 2, quad2, 0)
        for j in range(4):
            pltpu.make_async_copy(
                buf[4 + j], out_rows(B - 4 + j, 1), ws[4 + j]
            ).wait()

    return k(table, idx, pos)


def kernel(inputs, seq_len, embedding_table):
    B, S = inputs.shape
    V, D = embedding_table.shape
    pos = jnp.asarray(_positional_encoding(S, D))
    out = _sc_embed(embedding_table, inputs.astype(jnp.int32), pos)
    return out.reshape(B, S, D)


# speedup vs baseline: 1.0251x; 1.0251x over previous
import functools

import numpy as np
import jax
import jax.numpy as jnp
from jax import lax
from jax.experimental import pallas as pl
from jax.experimental.pallas import tpu as pltpu
from jax.experimental.pallas import tpu_sc as plsc

LANES = 16
NUM_CORES = 2
NUM_SUBCORES = 16
NUM_WORKERS = NUM_CORES * NUM_SUBCORES


def _positional_encoding(seq_len: int, d_model: int) -> np.ndarray:
    pos = np.arange(seq_len)[:, np.newaxis]
    i = np.arange(d_model)[np.newaxis, :]
    angle_rates = 1.0 / np.power(10000, 2 * (i // 2) / np.float32(d_model))
    angle_rads = pos * angle_rates
    angle_rads[:, 0::2] = np.sin(angle_rads[:, 0::2])
    angle_rads[:, 1::2] = np.cos(angle_rads[:, 1::2])
    return np.asarray(angle_rads, dtype=np.float32)


@jax.jit
def _sc_embed(table, idx, pos):
    S, D = pos.shape
    V, _ = table.shape
    B = idx.size // S
    SPW = S // NUM_WORKERS
    HP = SPW // 2
    NQ = B // 2
    scale = float(np.sqrt(np.float32(D)))

    mesh = plsc.VectorSubcoreMesh(core_axis_name="c", subcore_axis_name="s")

    @functools.partial(
        pl.kernel,
        mesh=mesh,
        out_type=jax.ShapeDtypeStruct((B * S, D), jnp.float32),
        scratch_types=[
            pltpu.VMEM((B * SPW,), jnp.int32),
            pltpu.VMEM((SPW, D), jnp.float32),
        ] + [pltpu.VMEM((HP, D), jnp.float32)] * 8
          + [pltpu.SemaphoreType.DMA] * 16,
    )
    def k(table_hbm, idx_hbm, pos_hbm, out_hbm, idx_v, pos_v, *bufs_and_sems):
        buf = bufs_and_sems[:8]
        gs = bufs_and_sems[8:16]
        ws = bufs_and_sems[16:24]
        wid = lax.axis_index("s") * NUM_CORES + lax.axis_index("c")
        s0 = wid * SPW

        pltpu.sync_copy(idx_hbm.at[wid], idx_v)
        pltpu.sync_copy(pos_hbm.at[pl.ds(s0, SPW)], pos_v)

        def idx_at(b, h):
            return idx_v.at[pl.ds(b * SPW + h * HP, HP)]

        def out_rows(b, h):
            return out_hbm.at[pl.ds(b * S + s0 + h * HP, HP)]

        for j in range(4):
            pltpu.async_copy(table_hbm.at[idx_at(j, 0)], buf[j], gs[j])

        def do_quad(qq, u):
            q = 2 * qq + u
            v = 1 - u
            b0 = 4 * qq
            nb0 = 4 * ((q + 1) // 2)
            for j in range(4):
                pltpu.make_async_copy(
                    table_hbm.at[idx_at(b0 + j, u)], buf[4 * u + j], gs[4 * u + j]
                ).wait()
            @pl.when(q >= 1)
            def _():
                pb0 = 4 * ((q - 1) // 2)
                for j in range(4):
                    pltpu.make_async_copy(
                        buf[4 * v + j], out_rows(pb0 + j, v), ws[4 * v + j]
                    ).wait()
            @pl.when(q + 1 < NQ)
            def _():
                for j in range(4):
                    pltpu.async_copy(
                        table_hbm.at[idx_at(nb0 + j, v)], buf[4 * v + j], gs[4 * v + j]
                    )

            def per_row(r, c):
                for kk in range(D // LANES):
                    sl = pl.ds(kk * LANES, LANES)
                    pv = pos_v[u * HP + r, sl]
                    for j in range(4):
                        bj = buf[4 * u + j]
                        bj[r, sl] = bj[r, sl] * scale + pv
                return c

            lax.fori_loop(0, HP, per_row, 0)
            for j in range(4):
                pltpu.async_copy(buf[4 * u + j], out_rows(b0 + j, u), ws[4 * u + j])

        def quad2(qq, c):
            do_quad(qq, 0)
            do_quad(qq, 1)
            return c

        lax.fori_loop(0, NQ // 2, quad2, 0)
        for j in range(4):
            pltpu.make_async_copy(
                buf[4 + j], out_rows(B - 4 + j, 1), ws[4 + j]
            ).wait()

    return k(table, idx, pos)


def kernel(inputs, seq_len, embedding_table):
    B, S = inputs.shape
    V, D = embedding_table.shape
    pos = jnp.asarray(_positional_encoding(S, D))
    spw = S // NUM_WORKERS
    idx = (inputs.astype(jnp.int32)
           .reshape(B, NUM_WORKERS, spw)
           .transpose(1, 0, 2)
           .reshape(NUM_WORKERS, B * spw))
    out = _sc_embed(embedding_table, idx, pos)
    return out.reshape(B, S, D)


# speedup vs baseline: 1.0321x; 1.0068x over previous
import functools

import numpy as np
import jax
import jax.numpy as jnp
from jax import lax
from jax.experimental import pallas as pl
from jax.experimental.pallas import tpu as pltpu
from jax.experimental.pallas import tpu_sc as plsc

LANES = 16
NUM_CORES = 2
NUM_SUBCORES = 16
NUM_WORKERS = NUM_CORES * NUM_SUBCORES


def _positional_encoding(seq_len: int, d_model: int) -> np.ndarray:
    pos = np.arange(seq_len)[:, np.newaxis]
    i = np.arange(d_model)[np.newaxis, :]
    angle_rates = 1.0 / np.power(10000, 2 * (i // 2) / np.float32(d_model))
    angle_rads = pos * angle_rates
    angle_rads[:, 0::2] = np.sin(angle_rads[:, 0::2])
    angle_rads[:, 1::2] = np.cos(angle_rads[:, 1::2])
    return np.asarray(angle_rads, dtype=np.float32)


@jax.jit
def _sc_embed(table, idx, pos):
    S, D = pos.shape
    V, _ = table.shape
    B = idx.size // S
    SPW = S // NUM_WORKERS
    HP = SPW // 2
    NQ = B // 2
    scale = float(np.sqrt(np.float32(D)))

    mesh = plsc.VectorSubcoreMesh(core_axis_name="c", subcore_axis_name="s")

    @functools.partial(
        pl.kernel,
        mesh=mesh,
        out_type=jax.ShapeDtypeStruct((B * S, D), jnp.float32),
        scratch_types=[
            pltpu.VMEM((B * SPW,), jnp.int32),
            pltpu.VMEM((SPW, D), jnp.float32),
        ] + [pltpu.VMEM((HP, D), jnp.float32)] * 8
          + [pltpu.SemaphoreType.DMA] * 16,
    )
    def k(table_hbm, idx_hbm, pos_hbm, out_hbm, idx_v, pos_v, *bufs_and_sems):
        buf = bufs_and_sems[:8]
        gs = bufs_and_sems[8:16]
        ws = bufs_and_sems[16:24]
        wid = lax.axis_index("s") * NUM_CORES + lax.axis_index("c")
        s0 = wid * SPW

        pltpu.sync_copy(idx_hbm.at[wid], idx_v)

        def idx_at(b, h):
            return idx_v.at[pl.ds(b * SPW + h * HP, HP)]

        def out_rows(b, h):
            return out_hbm.at[pl.ds(b * S + s0 + h * HP, HP)]

        for j in range(4):
            pltpu.async_copy(table_hbm.at[idx_at(j, 0)], buf[j], gs[j])
        pltpu.sync_copy(pos_hbm.at[pl.ds(s0, SPW)], pos_v)

        def do_quad(qq, u):
            q = 2 * qq + u
            v = 1 - u
            b0 = 4 * qq
            nb0 = 4 * ((q + 1) // 2)
            for j in range(4):
                pltpu.make_async_copy(
                    table_hbm.at[idx_at(b0 + j, u)], buf[4 * u + j], gs[4 * u + j]
                ).wait()
            @pl.when(q >= 1)
            def _():
                pb0 = 4 * ((q - 1) // 2)
                for j in range(4):
                    pltpu.make_async_copy(
                        buf[4 * v + j], out_rows(pb0 + j, v), ws[4 * v + j]
                    ).wait()
            @pl.when(q + 1 < NQ)
            def _():
                for j in range(4):
                    pltpu.async_copy(
                        table_hbm.at[idx_at(nb0 + j, v)], buf[4 * v + j], gs[4 * v + j]
                    )

            def per_row(r, c):
                for kk in range(D // LANES):
                    sl = pl.ds(kk * LANES, LANES)
                    pv = pos_v[u * HP + r, sl]
                    for j in range(4):
                        bj = buf[4 * u + j]
                        bj[r, sl] = bj[r, sl] * scale + pv
                return c

            lax.fori_loop(0, HP, per_row, 0)
            for j in range(4):
                pltpu.async_copy(buf[4 * u + j], out_rows(b0 + j, u), ws[4 * u + j])

        def quad2(qq, c):
            do_quad(qq, 0)
            do_quad(qq, 1)
            return c

        lax.fori_loop(0, NQ // 2, quad2, 0)
        for j in range(4):
            pltpu.make_async_copy(
                buf[4 + j], out_rows(B - 4 + j, 1), ws[4 + j]
            ).wait()

    return k(table, idx, pos)


def kernel(inputs, seq_len, embedding_table):
    B, S = inputs.shape
    V, D = embedding_table.shape
    pos = jnp.asarray(_positional_encoding(S, D))
    spw = S // NUM_WORKERS
    idx = (inputs.astype(jnp.int32)
           .reshape(B, NUM_WORKERS, spw)
           .transpose(1, 0, 2)
           .reshape(NUM_WORKERS, B * spw))
    out = _sc_embed(embedding_table, idx, pos)
    return out.reshape(B, S, D)
